# Initial kernel scaffold; baseline (speedup 1.0000x reference)
import jax
import jax.numpy as jnp
from jax.experimental import pallas as pl


def kernel(x, edge_index, W1, W2):
    raise NotImplementedError("write your pallas kernel here")


# speedup vs baseline: 9.4080x; 9.4080x over previous
import functools
import numpy as np

import jax
import jax.numpy as jnp
from jax import lax
from jax.experimental import pallas as pl
from jax.experimental.pallas import tpu as pltpu
from jax.experimental.pallas import tpu_sc as plsc

ALPHA = 0.5
THETA = 1.0

NC = 2
NS = 16
NW = NC * NS
CHUNK = 128
DUMMY = 16
ROW_BLOCK = 2000


def _sc_degree(dstp, npad, nchunk):
    deg_tile = npad // NS
    mesh = plsc.VectorSubcoreMesh(core_axis_name="c", subcore_axis_name="s")

    @functools.partial(
        pl.kernel,
        mesh=mesh,
        out_type=jax.ShapeDtypeStruct((NC, npad), jnp.float32),
        scratch_types=[
            pltpu.VMEM((nchunk, CHUNK), jnp.int32),
            pltpu.VMEM((deg_tile,), jnp.float32),
            pltpu.VMEM((CHUNK,), jnp.float32),
            pltpu.VMEM_SHARED((npad,), jnp.float32),
        ],
    )
    def deg_kernel(dst_hbm, out_hbm, idx_v, zbuf, ones_v, acc):
        c = lax.axis_index("c")
        s = lax.axis_index("s")
        w = c * NS + s
        pltpu.sync_copy(dst_hbm.at[w], idx_v)

        @pl.loop(0, deg_tile, step=16)
        def _(i):
            zbuf[pl.ds(i, 16)] = jnp.zeros((16,), jnp.float32)

        @pl.loop(0, CHUNK, step=16)
        def _(i):
            ones_v[pl.ds(i, 16)] = jnp.ones((16,), jnp.float32)

        pltpu.sync_copy(zbuf, acc.at[pl.ds(s * deg_tile, deg_tile)])
        plsc.subcore_barrier()

        @pl.loop(0, nchunk)
        def _(j):
            pltpu.sync_copy(ones_v, acc.at[idx_v.at[j]], add=True)

        plsc.subcore_barrier()
        pltpu.sync_copy(acc.at[pl.ds(s * deg_tile, deg_tile)],
                        out_hbm.at[c, pl.ds(s * deg_tile, deg_tile)])

    return deg_kernel(dstp)


def _sc_msgpass(g, srcp, dstp, npad, nchunk):
    _, d = g.shape
    rows_tile = npad // NS
    mesh = plsc.VectorSubcoreMesh(core_axis_name="c", subcore_axis_name="s")

    @functools.partial(
        pl.kernel,
        mesh=mesh,
        out_type=jax.ShapeDtypeStruct((NC, npad, d), jnp.float32),
        scratch_types=[
            pltpu.VMEM((nchunk, CHUNK), jnp.int32),
            pltpu.VMEM((nchunk, CHUNK), jnp.int32),
            pltpu.VMEM((CHUNK, d), jnp.float32),
            pltpu.VMEM_SHARED((npad, d), jnp.float32),
        ],
    )
    def msg_kernel(g_hbm, src_hbm, dst_hbm, out_hbm, src_v, dst_v, buf, acc):
        c = lax.axis_index("c")
        s = lax.axis_index("s")
        w = c * NS + s
        pltpu.sync_copy(src_hbm.at[w], src_v)
        pltpu.sync_copy(dst_hbm.at[w], dst_v)

        @pl.loop(0, CHUNK)
        def _(r):
            @pl.loop(0, d, step=16)
            def _(col):
                buf[r, pl.ds(col, 16)] = jnp.zeros((16,), jnp.float32)

        base = s * rows_tile

        @pl.loop(0, rows_tile, step=CHUNK)
        def _(r):
            pltpu.sync_copy(buf, acc.at[pl.ds(base + r, CHUNK)])

        plsc.subcore_barrier()

        @pl.loop(0, nchunk)
        def _(j):
            pltpu.sync_copy(g_hbm.at[src_v.at[j]], buf)
            pltpu.sync_copy(buf, acc.at[dst_v.at[j]], add=True)

        plsc.subcore_barrier()

        @pl.loop(0, rows_tile, step=CHUNK)
        def _(r):
            pltpu.sync_copy(acc.at[pl.ds(base + r, CHUNK)],
                            out_hbm.at[c, pl.ds(base + r, CHUNK)])

    return msg_kernel(g, srcp, dstp)


def _tc_prep(deg3, x):
    n, d = x.shape

    def body(deg_ref, x_ref, dinv_ref, g_ref):
        deg = deg_ref[0] + deg_ref[1] + 1.0
        dinv = lax.rsqrt(deg)
        dinv_ref[...] = dinv
        g_ref[...] = dinv * x_ref[...]

    grid = (n // ROW_BLOCK,)
    return pl.pallas_call(
        body,
        grid=grid,
        in_specs=[
            pl.BlockSpec((NC, ROW_BLOCK, 1), lambda r: (0, r, 0)),
            pl.BlockSpec((ROW_BLOCK, d), lambda r: (r, 0)),
        ],
        out_specs=[
            pl.BlockSpec((ROW_BLOCK, 1), lambda r: (r, 0)),
            pl.BlockSpec((ROW_BLOCK, d), lambda r: (r, 0)),
        ],
        out_shape=[
            jax.ShapeDtypeStruct((n, 1), jnp.float32),
            jax.ShapeDtypeStruct((n, d), jnp.float32),
        ],
    )(deg3, x)


def _tc_layer(parts, g, x, dinv, a, b, last):
    n, d = x.shape

    def body(p_ref, g_ref, x_ref, dinv_ref, a_ref, b_ref, o_ref):
        t = (p_ref[0] + p_ref[1] + g_ref[...]) * dinv_ref[...]
        out = jnp.dot(t, a_ref[...], precision=lax.Precision.HIGHEST,
                      preferred_element_type=jnp.float32)
        out += jnp.dot(x_ref[...], b_ref[...], precision=lax.Precision.HIGHEST,
                       preferred_element_type=jnp.float32)
        h = jnp.maximum(out, 0.0)
        o_ref[...] = h if last else h * dinv_ref[...]

    grid = (n // ROW_BLOCK,)
    return pl.pallas_call(
        body,
        grid=grid,
        in_specs=[
            pl.BlockSpec((NC, ROW_BLOCK, d), lambda r: (0, r, 0)),
            pl.BlockSpec((ROW_BLOCK, d), lambda r: (r, 0)),
            pl.BlockSpec((ROW_BLOCK, d), lambda r: (r, 0)),
            pl.BlockSpec((ROW_BLOCK, 1), lambda r: (r, 0)),
            pl.BlockSpec((d, d), lambda r: (0, 0)),
            pl.BlockSpec((d, d), lambda r: (0, 0)),
        ],
        out_specs=pl.BlockSpec((ROW_BLOCK, d), lambda r: (r, 0)),
        out_shape=jax.ShapeDtypeStruct((n, d), jnp.float32),
    )(parts, g, x, dinv, a, b)


def kernel(x, edge_index, W1, W2):
    n, d = x.shape
    e = edge_index.shape[1]
    layers = W1.shape[0]
    assert n % ROW_BLOCK == 0 and d % 128 == 0

    epad = -(-e // (NW * CHUNK)) * (NW * CHUNK)
    nchunk = epad // (NW * CHUNK)
    npad = -(-(n + DUMMY) // (NS * CHUNK)) * (NS * CHUNK)
    src = edge_index[0]
    dst = edge_index[1]
    pad = epad - e
    srcp = jnp.concatenate(
        [src, jnp.zeros((pad,), jnp.int32)]).reshape(NW, nchunk, CHUNK)
    dstp = jnp.concatenate(
        [dst, n + (jnp.arange(pad, dtype=jnp.int32) % DUMMY)]
    ).reshape(NW, nchunk, CHUNK)

    betas = np.log(THETA / (np.arange(1, layers + 1)) + 1.0).astype(np.float32)
    eye = jnp.eye(d, dtype=jnp.float32)
    bet = jnp.asarray(betas)[:, None, None]
    a_all = (1.0 - ALPHA) * ((1.0 - bet) * eye + bet * W1)
    b_all = ALPHA * ((1.0 - bet) * eye + bet * W2)

    deg_parts = _sc_degree(dstp, npad, nchunk)
    dinv, g = _tc_prep(deg_parts.reshape(NC, npad, 1), x)
    for i in range(layers):
        parts = _sc_msgpass(g, srcp, dstp, npad, nchunk)
        g = _tc_layer(parts, g, x, dinv, a_all[i], b_all[i],
                      last=(i == layers - 1))
    return g
